# baseline (speedup 1.0000x reference)
import functools

import jax
import jax.numpy as jnp
from jax import lax
from jax.experimental import pallas as pl
from jax.experimental.pallas import tpu as pltpu
from jax.experimental.pallas import tpu_sc as plsc

VOCAB = 100000
EMBED_DIM = 32
GENRE_DIM = 19
BATCH = 16384


def _sc_gather(emb_table, movie_id):
  info = plsc.get_sparse_core_info()
  nc, ns = info.num_cores, info.num_subcores
  nw = nc * ns
  b_per_w = BATCH // nw

  mesh = plsc.VectorSubcoreMesh(core_axis_name="c", subcore_axis_name="s")

  @functools.partial(
      pl.kernel,
      mesh=mesh,
      compiler_params=pltpu.CompilerParams(use_tc_tiling_on_sc=False),
      out_type=jax.ShapeDtypeStruct((BATCH, EMBED_DIM), jnp.float32),
      scratch_types=[
          pltpu.VMEM((b_per_w,), jnp.int32),
          pltpu.VMEM((b_per_w, EMBED_DIM), jnp.float32),
          pltpu.SemaphoreType.DMA,
      ],
  )
  def gather_kernel(table_hbm, idx_hbm, out_hbm, idx_v, rows_v, sem):
    wid = lax.axis_index("s") * nc + lax.axis_index("c")
    base = wid * b_per_w
    pltpu.sync_copy(idx_hbm.at[pl.ds(base, b_per_w)], idx_v)
    pltpu.async_copy(table_hbm.at[idx_v], rows_v, sem).wait()
    pltpu.sync_copy(rows_v, out_hbm.at[pl.ds(base, b_per_w)])

  return gather_kernel(emb_table, movie_id)


def _mlp_body(me_ref, gp_ref, gW1_ref, gb1_ref, gW2_ref, gb2_ref,
              rW1_ref, rb1_ref, rW2_ref, rb2_ref, rW3_ref, rb3_ref,
              out_ref):
  gp = gp_ref[...]
  h = jnp.maximum(
      jnp.dot(gp, gW1_ref[...], preferred_element_type=jnp.float32)
      + gb1_ref[...], 0.0)
  ge = jnp.dot(h, gW2_ref[...], preferred_element_type=jnp.float32) + gb2_ref[...]
  combined = jnp.concatenate([me_ref[...], ge], axis=1)
  h = jnp.maximum(
      jnp.dot(combined, rW1_ref[...], preferred_element_type=jnp.float32)
      + rb1_ref[...], 0.0)
  h = jnp.maximum(
      jnp.dot(h, rW2_ref[...], preferred_element_type=jnp.float32)
      + rb2_ref[...], 0.0)
  out_ref[...] = (
      jnp.dot(h, rW3_ref[...], preferred_element_type=jnp.float32)
      + rb3_ref[...])


def _tc_mlp(movie_emb, genre_preferences, gW1, gb1, gW2, gb2,
            rW1, rb1, rW2, rb2, rW3, rb3, block_b=2048):
  grid = (BATCH // block_b,)

  def rows(i):
    return (i, 0)

  def whole(i):
    return (0, 0)

  full = lambda a: pl.BlockSpec(a.shape, whole)
  return pl.pallas_call(
      _mlp_body,
      grid=grid,
      in_specs=[
          pl.BlockSpec((block_b, EMBED_DIM), rows),
          pl.BlockSpec((block_b, GENRE_DIM), rows),
          full(gW1), full(gb1), full(gW2), full(gb2),
          full(rW1), full(rb1), full(rW2), full(rb2), full(rW3), full(rb3),
      ],
      out_specs=pl.BlockSpec((block_b, 1), rows),
      out_shape=jax.ShapeDtypeStruct((BATCH, 1), jnp.float32),
  )(movie_emb, genre_preferences, gW1, gb1, gW2, gb2,
    rW1, rb1, rW2, rb2, rW3, rb3)


@jax.jit
def kernel(movieId, genre_preferences, emb_table, gW1, gb1, gW2, gb2,
           rW1, rb1, rW2, rb2, rW3, rb3):
  movie_emb = _sc_gather(emb_table, movieId.astype(jnp.int32))
  return _tc_mlp(
      movie_emb, genre_preferences,
      gW1, gb1.reshape(1, -1), gW2, gb2.reshape(1, -1),
      rW1, rb1.reshape(1, -1), rW2, rb2.reshape(1, -1),
      rW3, rb3.reshape(1, -1))
